# Initial kernel scaffold; baseline (speedup 1.0000x reference)
import jax
import jax.numpy as jnp
from jax.experimental import pallas as pl


def kernel(p, mesh):
    raise NotImplementedError("write your pallas kernel here")


# speedup vs baseline: 5377.8256x; 5377.8256x over previous
import jax
import jax.numpy as jnp
from jax.experimental import pallas as pl

_RES = 384
_D_S = 2
_D_F = 8
_KERNEL_SIGMA = 0.005
_FEATURE_SIGMA = 0.02
_N = 2048
_HW = int(round(3 * _KERNEL_SIGMA * _RES))
_HWF = int(round(3 * _FEATURE_SIGMA * _RES))


def _splat_kernel(p_ref, xs_ref, ys_ref, out_ref):
    p = p_ref[...]
    cx = p[:, 0:1]
    cy = p[:, 1:2]
    xs = xs_ref[...]
    ys = ys_ref[...]

    coor_x = jnp.floor(cx * _RES).astype(jnp.int32)
    coor_y = jnp.floor(cy * _RES).astype(jnp.int32)
    idx = jax.lax.broadcasted_iota(jnp.int32, (1, _RES), 1)

    def profiles(center, coor, grid, hw, sigma):
        d = grid - center
        g = jnp.exp(d * d * (-1.0 / (2.0 * sigma * sigma)))
        mask = (idx >= coor - hw) & (idx <= coor + hw)
        return jnp.where(mask, g, 0.0)

    dn = (((0,), (0,)), ((), ()))
    hi = jax.lax.Precision.HIGHEST

    gy0 = profiles(cy, coor_y, ys, _HW, _KERNEL_SIGMA)
    gx0 = profiles(cx, coor_x, xs, _HW, _KERNEL_SIGMA)
    out_ref[0, :, :] = jax.lax.dot_general(
        gy0, gx0, dn, precision=hi, preferred_element_type=jnp.float32)

    gyf = profiles(cy, coor_y, ys, _HWF, _FEATURE_SIGMA)
    gxf = profiles(cx, coor_x, xs, _HWF, _FEATURE_SIGMA)
    for f in range(_D_F):
        w = p[:, _D_S + f : _D_S + f + 1] * gxf
        out_ref[1 + f, :, :] = jax.lax.dot_general(
            gyf, w, dn, precision=hi, preferred_element_type=jnp.float32)


def kernel(p, mesh):
    xs = mesh[0, 0:1, :]
    ys = mesh[1, :, 0]
    ys = ys[None, :]
    out = pl.pallas_call(
        _splat_kernel,
        out_shape=jax.ShapeDtypeStruct((_D_F + 1, _RES, _RES), jnp.float32),
    )(p, xs, ys)
    return out[None]


# speedup vs baseline: 14508.4359x; 2.6978x over previous
import jax
import jax.numpy as jnp
from jax.experimental import pallas as pl

_RES = 384
_D_S = 2
_D_F = 8
_KERNEL_SIGMA = 0.005
_FEATURE_SIGMA = 0.02
_N = 2048
_HW = int(round(3 * _KERNEL_SIGMA * _RES))
_HWF = int(round(3 * _FEATURE_SIGMA * _RES))


def _splat_kernel(p_ref, xs_ref, ys_ref, out_ref):
    p = p_ref[...]
    cx = p[:, 0:1]
    cy = p[:, 1:2]
    xs = xs_ref[...]
    ys = ys_ref[...]

    coor_x = jnp.floor(cx * _RES).astype(jnp.int32)
    coor_y = jnp.floor(cy * _RES).astype(jnp.int32)
    idx = jax.lax.broadcasted_iota(jnp.int32, (1, _RES), 1)

    def raw(center, grid, sigma):
        d = grid - center
        return jnp.exp(d * d * (-1.0 / (2.0 * sigma * sigma)))

    def window(coor, hw):
        return (idx >= coor - hw) & (idx <= coor + hw)

    def pow16(g):
        g2 = g * g
        g4 = g2 * g2
        g8 = g4 * g4
        return g8 * g8

    ey = raw(cy, ys, _FEATURE_SIGMA)
    ex = raw(cx, xs, _FEATURE_SIGMA)
    gyf = jnp.where(window(coor_y, _HWF), ey, 0.0)
    gxf = jnp.where(window(coor_x, _HWF), ex, 0.0)
    gy0 = jnp.where(window(coor_y, _HW), pow16(ey), 0.0)
    gx0 = jnp.where(window(coor_x, _HW), pow16(ex), 0.0)

    dn = (((0,), (0,)), ((), ()))
    prec = jax.lax.Precision.DEFAULT

    out_ref[0, :, :] = jax.lax.dot_general(
        gy0, gx0, dn, precision=prec, preferred_element_type=jnp.float32)

    for f in range(_D_F):
        w = p[:, _D_S + f : _D_S + f + 1] * gxf
        out_ref[1 + f, :, :] = jax.lax.dot_general(
            gyf, w, dn, precision=prec, preferred_element_type=jnp.float32)


def kernel(p, mesh):
    xs = mesh[0, 0:1, :]
    ys = mesh[1, :, 0]
    ys = ys[None, :]
    out = pl.pallas_call(
        _splat_kernel,
        out_shape=jax.ShapeDtypeStruct((_D_F + 1, _RES, _RES), jnp.float32),
    )(p, xs, ys)
    return out[None]


# speedup vs baseline: 14658.3683x; 1.0103x over previous
import jax
import jax.numpy as jnp
from jax.experimental import pallas as pl

_RES = 384
_D_S = 2
_D_F = 8
_KERNEL_SIGMA = 0.005
_FEATURE_SIGMA = 0.02
_N = 2048
_HW = int(round(3 * _KERNEL_SIGMA * _RES))
_HWF = int(round(3 * _FEATURE_SIGMA * _RES))


def _splat_kernel(p_ref, xs_ref, ys_ref, out_ref):
    p = p_ref[...]
    cx = p[:, 0:1]
    cy = p[:, 1:2]
    xs = xs_ref[...]
    ys = ys_ref[...]

    coor_x = jnp.floor(cx * _RES).astype(jnp.int32)
    coor_y = jnp.floor(cy * _RES).astype(jnp.int32)
    idx = jax.lax.broadcasted_iota(jnp.int32, (1, _RES), 1)

    def raw(center, grid, sigma):
        d = grid - center
        return jnp.exp(d * d * (-1.0 / (2.0 * sigma * sigma)))

    def window(coor, hw):
        return (idx >= coor - hw) & (idx <= coor + hw)

    def pow16(g):
        g2 = g * g
        g4 = g2 * g2
        g8 = g4 * g4
        return g8 * g8

    bf16 = jnp.bfloat16
    ey = raw(cy, ys, _FEATURE_SIGMA)
    ex = raw(cx, xs, _FEATURE_SIGMA)
    gyf = jnp.where(window(coor_y, _HWF), ey, 0.0).astype(bf16)
    gxf = jnp.where(window(coor_x, _HWF), ex, 0.0).astype(bf16)
    gy0 = jnp.where(window(coor_y, _HW), pow16(ey), 0.0).astype(bf16)
    gx0 = jnp.where(window(coor_x, _HW), pow16(ex), 0.0).astype(bf16)

    dn = (((0,), (0,)), ((), ()))
    prec = jax.lax.Precision.DEFAULT

    out_ref[0, :, :] = jax.lax.dot_general(
        gy0, gx0, dn, precision=prec, preferred_element_type=jnp.float32)

    for f in range(_D_F):
        w = p[:, _D_S + f : _D_S + f + 1].astype(bf16) * gxf
        out_ref[1 + f, :, :] = jax.lax.dot_general(
            gyf, w, dn, precision=prec, preferred_element_type=jnp.float32)


def kernel(p, mesh):
    xs = mesh[0, 0:1, :]
    ys = mesh[1, :, 0]
    ys = ys[None, :]
    out = pl.pallas_call(
        _splat_kernel,
        out_shape=jax.ShapeDtypeStruct((_D_F + 1, _RES, _RES), jnp.float32),
    )(p, xs, ys)
    return out[None]


# speedup vs baseline: 15925.5402x; 1.0864x over previous
import jax
import jax.numpy as jnp
from jax.experimental import pallas as pl

_RES = 384
_D_S = 2
_D_F = 8
_KERNEL_SIGMA = 0.005
_FEATURE_SIGMA = 0.02
_N = 2048
_HW = int(round(3 * _KERNEL_SIGMA * _RES))
_HWF = int(round(3 * _FEATURE_SIGMA * _RES))


def _splat_kernel(p_ref, xs_ref, ys_ref, out_ref):
    p = p_ref[...]
    cx = p[:, 0:1]
    cy = p[:, 1:2]
    xs = xs_ref[...]
    ys = ys_ref[...]

    coor_x = jnp.floor(cx * _RES).astype(jnp.int32)
    coor_y = jnp.floor(cy * _RES).astype(jnp.int32)
    idx = jax.lax.broadcasted_iota(jnp.int32, (1, _RES), 1)

    def raw(center, grid, sigma):
        d = grid - center
        return jnp.exp(d * d * (-1.0 / (2.0 * sigma * sigma)))

    def window(coor, hw):
        return (idx >= coor - hw) & (idx <= coor + hw)

    def pow16(g):
        g2 = g * g
        g4 = g2 * g2
        g8 = g4 * g4
        return g8 * g8

    bf16 = jnp.bfloat16
    ey = raw(cy, ys, _FEATURE_SIGMA)
    ex = raw(cx, xs, _FEATURE_SIGMA)
    gyf = jnp.where(window(coor_y, _HWF), ey, 0.0).astype(bf16)
    gxf = jnp.where(window(coor_x, _HWF), ex, 0.0).astype(bf16)
    gy0 = jnp.where(window(coor_y, _HW), pow16(ey), 0.0).astype(bf16)
    gx0 = jnp.where(window(coor_x, _HW), pow16(ex), 0.0).astype(bf16)

    dn = (((0,), (0,)), ((), ()))
    prec = jax.lax.Precision.DEFAULT

    out_ref[0, :, :] = jax.lax.dot_general(
        gy0, gx0, dn, precision=prec, preferred_element_type=jnp.float32)

    w = jnp.concatenate(
        [p[:, _D_S + f : _D_S + f + 1].astype(bf16) * gxf for f in range(_D_F)],
        axis=1)
    res = jax.lax.dot_general(
        gyf, w, dn, precision=prec, preferred_element_type=jnp.float32)
    for f in range(_D_F):
        out_ref[1 + f, :, :] = res[:, f * _RES:(f + 1) * _RES]


def kernel(p, mesh):
    xs = mesh[0, 0:1, :]
    ys = mesh[1, :, 0]
    ys = ys[None, :]
    out = pl.pallas_call(
        _splat_kernel,
        out_shape=jax.ShapeDtypeStruct((_D_F + 1, _RES, _RES), jnp.float32),
    )(p, xs, ys)
    return out[None]


# speedup vs baseline: 16468.4865x; 1.0341x over previous
import jax
import jax.numpy as jnp
from jax.experimental import pallas as pl

_RES = 384
_D_S = 2
_D_F = 8
_KERNEL_SIGMA = 0.005
_FEATURE_SIGMA = 0.02
_N = 2048
_HW = int(round(3 * _KERNEL_SIGMA * _RES))
_HWF = int(round(3 * _FEATURE_SIGMA * _RES))


def _splat_kernel(p_ref, pt_ref, xs_ref, ys_ref, out_ref):
    bf16 = jnp.bfloat16
    p = p_ref[...]
    pt = pt_ref[...]
    xs = xs_ref[...]
    ysc = ys_ref[...]

    kf = -1.0 / (2.0 * _FEATURE_SIGMA * _FEATURE_SIGMA)
    k0 = -1.0 / (2.0 * _KERNEL_SIGMA * _KERNEL_SIGMA)

    cyr = pt[1:2, :]
    coor_yr = jnp.floor(cyr * _RES).astype(jnp.int32)
    ridx = jax.lax.broadcasted_iota(jnp.int32, (_RES, 1), 0)
    dy = ysc - cyr
    dy2 = dy * dy
    myf = (ridx >= coor_yr - _HWF) & (ridx <= coor_yr + _HWF)
    my0 = (ridx >= coor_yr - _HW) & (ridx <= coor_yr + _HW)
    gyf = jnp.where(myf, jnp.exp(dy2 * kf), 0.0).astype(bf16)
    gy0 = jnp.where(my0, jnp.exp(dy2 * k0), 0.0).astype(bf16)

    cxc = p[:, 0:1]
    coor_xc = jnp.floor(cxc * _RES).astype(jnp.int32)
    cidx = jax.lax.broadcasted_iota(jnp.int32, (1, _RES), 1)
    dx = xs - cxc
    dx2 = dx * dx
    mxf = (cidx >= coor_xc - _HWF) & (cidx <= coor_xc + _HWF)
    mx0 = (cidx >= coor_xc - _HW) & (cidx <= coor_xc + _HW)
    gxf = jnp.where(mxf, jnp.exp(dx2 * kf), 0.0).astype(bf16)
    gx0 = jnp.where(mx0, jnp.exp(dx2 * k0), 0.0).astype(bf16)

    dn = (((1,), (0,)), ((), ()))
    prec = jax.lax.Precision.DEFAULT

    out_ref[0, :, :] = jax.lax.dot_general(
        gy0, gx0, dn, precision=prec, preferred_element_type=jnp.float32)

    lhs = jnp.concatenate(
        [pt[_D_S + f:_D_S + f + 1, :].astype(bf16) * gyf for f in range(_D_F)],
        axis=0)
    res = jax.lax.dot_general(
        lhs, gxf, dn, precision=prec, preferred_element_type=jnp.float32)
    for f in range(_D_F):
        out_ref[1 + f, :, :] = res[f * _RES:(f + 1) * _RES, :]


def kernel(p, mesh):
    xs = mesh[0, 0:1, :]
    ys = mesh[1, :, 0:1]
    pt = p.T
    out = pl.pallas_call(
        _splat_kernel,
        out_shape=jax.ShapeDtypeStruct((_D_F + 1, _RES, _RES), jnp.float32),
    )(p, pt, xs, ys)
    return out[None]
